# Initial kernel scaffold; baseline (speedup 1.0000x reference)
import jax
import jax.numpy as jnp
from jax.experimental import pallas as pl


def kernel(x, edge_index, segment_ids, params):
    raise NotImplementedError("write your pallas kernel here")


# speedup vs baseline: 7.2192x; 7.2192x over previous
import functools

import jax
import jax.numpy as jnp
from jax import lax
from jax.experimental import pallas as pl
from jax.experimental.pallas import tpu as pltpu
from jax.experimental.pallas import tpu_sc as plsc

N = 10000
E = 320000
D = 128
H = 64
G = 64
BN_EPS = 1e-3
EPS = 1e-5

_NC = 2
_NS = 16
_LANE = 64
_NW = _NC * _NS
_RPW = 160
_ROWS = _RPW * _NW
_EPAD = _ROWS * _LANE - E
_NACC = N + 8
_TPT = 632
_TPT_LAST = _NACC - 15 * _TPT
_WB_LAST = N - 15 * _TPT

_BLK = 2000
_NBLK = N // _BLK


def _make_edge_agg(W):
  mesh = plsc.VectorSubcoreMesh(core_axis_name="c", subcore_axis_name="s",
                                num_cores=_NC, num_subcores=_NS)

  @functools.partial(
      pl.kernel,
      mesh=mesh,
      out_type=jax.ShapeDtypeStruct((_NC, N, W), jnp.float32),
      scratch_types=[
          pltpu.VMEM((_RPW, _LANE), jnp.int32),
          pltpu.VMEM((_RPW, _LANE), jnp.int32),
          pltpu.VMEM((2, _LANE, W), jnp.float32),
          pltpu.VMEM_SHARED((_NACC, W), jnp.float32),
          pltpu.SemaphoreType.DMA,
      ],
      name="edge_agg_w%d" % W,
      compiler_params=pltpu.CompilerParams(use_tc_tiling_on_sc=False),
  )
  def edge_agg(hw_hbm, col_hbm, row_hbm, zero_hbm, out_hbm,
               col_v, row_v, gbuf, acc, sem):
    c = lax.axis_index("c")
    s = lax.axis_index("s")
    wid = c * _NS + s

    @pl.when(s < _NS - 1)
    def _():
      pltpu.sync_copy(zero_hbm.at[pl.ds(s * _TPT, _TPT)],
                      acc.at[pl.ds(s * _TPT, _TPT)])

    @pl.when(s == _NS - 1)
    def _():
      pltpu.sync_copy(zero_hbm.at[pl.ds(15 * _TPT, _TPT_LAST)],
                      acc.at[pl.ds(15 * _TPT, _TPT_LAST)])

    base = wid * _RPW
    pltpu.sync_copy(col_hbm.at[pl.ds(base, _RPW)], col_v)
    pltpu.sync_copy(row_hbm.at[pl.ds(base, _RPW)], row_v)

    plsc.subcore_barrier()

    def do_chunk(j, parity):
      pltpu.async_copy(hw_hbm.at[col_v.at[j]], gbuf.at[parity], sem).wait()
      pltpu.sync_copy(gbuf.at[parity], acc.at[row_v.at[j]], add=True)

    def body(i, carry):
      do_chunk(2 * i, 0)
      do_chunk(2 * i + 1, 1)
      return carry

    lax.fori_loop(0, _RPW // 2, body, 0)

    plsc.subcore_barrier()

    @pl.when(s < _NS - 1)
    def _():
      pltpu.sync_copy(acc.at[pl.ds(s * _TPT, _TPT)],
                      out_hbm.at[c, pl.ds(s * _TPT, _TPT)])

    @pl.when(s == _NS - 1)
    def _():
      pltpu.sync_copy(acc.at[pl.ds(15 * _TPT, _WB_LAST)],
                      out_hbm.at[c, pl.ds(15 * _TPT, _WB_LAST)])

  return edge_agg


@functools.lru_cache(maxsize=None)
def _edge_agg(W):
  return _make_edge_agg(W)


def _mm_body(x_ref, w_ref, o_ref):
  o_ref[...] = jnp.dot(x_ref[...], w_ref[...],
                       preferred_element_type=jnp.float32)


def _matmul(x, w):
  m, k = x.shape
  _, n = w.shape
  return pl.pallas_call(
      _mm_body,
      grid=(_NBLK,),
      in_specs=[pl.BlockSpec((_BLK, k), lambda i: (i, 0)),
                pl.BlockSpec((k, n), lambda i: (0, 0))],
      out_specs=pl.BlockSpec((_BLK, n), lambda i: (i, 0)),
      out_shape=jax.ShapeDtypeStruct((m, n), jnp.float32),
  )(x, w)


def _relu_mm_body(a_ref, b_ref, w_ref, o_ref):
  h = jnp.maximum(a_ref[0] + a_ref[1] + b_ref[...], 0.0)
  o_ref[...] = jnp.dot(h, w_ref[...], preferred_element_type=jnp.float32)


def _relu_matmul(agg, b, w):
  k = agg.shape[2]
  n = w.shape[1]
  return pl.pallas_call(
      _relu_mm_body,
      grid=(_NBLK,),
      in_specs=[pl.BlockSpec((_NC, _BLK, k), lambda i: (0, i, 0)),
                pl.BlockSpec((1, k), lambda i: (0, 0)),
                pl.BlockSpec((k, n), lambda i: (0, 0))],
      out_specs=pl.BlockSpec((_BLK, n), lambda i: (i, 0)),
      out_shape=jax.ShapeDtypeStruct((N, n), jnp.float32),
  )(agg, b, w)


def _pool_body(x_ref, agg_ref, b_ref, seg_ref, z_ref,
               cnt, xsum, xsq, xmax, xmin, hmax, hsum):
  i = pl.program_id(0)

  @pl.when(i == 0)
  def _():
    cnt[...] = jnp.zeros_like(cnt)
    xsum[...] = jnp.zeros_like(xsum)
    xsq[...] = jnp.zeros_like(xsq)
    hsum[...] = jnp.zeros_like(hsum)
    xmax[...] = jnp.full_like(xmax, -jnp.inf)
    xmin[...] = jnp.full_like(xmin, jnp.inf)
    hmax[...] = jnp.full_like(hmax, -jnp.inf)

  x = x_ref[...]
  h = jnp.maximum(agg_ref[0] + agg_ref[1] + b_ref[...], 0.0)
  seg = seg_ref[...]
  gio = lax.broadcasted_iota(jnp.int32, (_BLK, G), 1)
  onehot = (seg == gio).astype(jnp.float32)
  ones = jnp.ones((_BLK, D), jnp.float32)
  dn = (((0,), (0,)), ((), ()))
  cnt[...] += lax.dot_general(onehot, ones, dn,
                              preferred_element_type=jnp.float32)
  xsum[...] += lax.dot_general(onehot, x, dn,
                               preferred_element_type=jnp.float32)
  xsq[...] += lax.dot_general(onehot, x * x, dn,
                              preferred_element_type=jnp.float32)
  hsum[...] += lax.dot_general(onehot, h, dn,
                               preferred_element_type=jnp.float32)

  g_lo = seg_ref[0, 0]
  g_hi = seg_ref[_BLK - 1, 0]
  rio = lax.broadcasted_iota(jnp.int32, (G, 1), 0)

  def gbody(g, carry):
    m = seg == g
    mx = jnp.max(jnp.where(m, x, -jnp.inf), axis=0, keepdims=True)
    mn = jnp.min(jnp.where(m, x, jnp.inf), axis=0, keepdims=True)
    mh = jnp.max(jnp.where(m, h, -jnp.inf), axis=0, keepdims=True)
    rs = rio == g
    xmax[...] = jnp.where(rs, jnp.maximum(xmax[...], mx), xmax[...])
    xmin[...] = jnp.where(rs, jnp.minimum(xmin[...], mn), xmin[...])
    hmax[...] = jnp.where(rs, jnp.maximum(hmax[...], mh), hmax[...])
    return carry

  lax.fori_loop(g_lo, g_hi + 1, gbody, 0)

  @pl.when(i == _NBLK - 1)
  def _():
    c = jnp.maximum(cnt[...], 1.0)
    avg = xsum[...] / c
    z_ref[:, 0:128] = hmax[...]
    z_ref[:, 128:256] = hsum[...] / c
    z_ref[:, 256:384] = hsum[...]
    z_ref[:, 384:512] = avg
    z_ref[:, 512:640] = jnp.abs(xsq[...] / c - avg * avg)
    z_ref[:, 640:768] = xmax[...]
    z_ref[:, 768:896] = xmin[...]


def _pool(x, agg, b, seg2):
  return pl.pallas_call(
      _pool_body,
      grid=(_NBLK,),
      in_specs=[pl.BlockSpec((_BLK, D), lambda i: (i, 0)),
                pl.BlockSpec((_NC, _BLK, D), lambda i: (0, i, 0)),
                pl.BlockSpec((1, D), lambda i: (0, 0)),
                pl.BlockSpec((_BLK, 1), lambda i: (i, 0))],
      out_specs=pl.BlockSpec((G, 896), lambda i: (0, 0)),
      out_shape=jax.ShapeDtypeStruct((G, 896), jnp.float32),
      scratch_shapes=[pltpu.VMEM((G, D), jnp.float32)] * 7,
  )(x, agg, b, seg2)


def _dec_body(z_ref, *refs):
  (d0w, d0b, g0, be0, m0, v0,
   d1w, d1b, g1, be1, m1, v1,
   d2w, d2b, g2, be2, m2, v2,
   l0w, l0b, l1w, l1b, l2w, l2b,
   a0w, a0b, a1w, a1b, a2w, a2b, asw, asb,
   s0w, s0b, s1w, s1b, s2w, s2b, o_ref) = refs

  z = z_ref[...]
  for (w, b, gm, bt, mu, vr) in ((d0w, d0b, g0, be0, m0, v0),
                                 (d1w, d1b, g1, be1, m1, v1),
                                 (d2w, d2b, g2, be2, m2, v2)):
    z = jnp.dot(z, w[...], preferred_element_type=jnp.float32) + b[...]
    z = jnp.where(z >= 0, z, 0.15 * z)
    z = ((z - mu[...]) / jnp.sqrt(vr[...] + BN_EPS)) * gm[...] + bt[...]

  def dense(t, w, b):
    return jnp.dot(t, w[...], preferred_element_type=jnp.float32) + b[...]

  x_loge = dense(dense(dense(z, l0w, l0b), l1w, l1b), l2w, l2b)
  x_ang = dense(dense(dense(z, a0w, a0b), a1w, a1b), a2w, a2b)
  zeniazi = jax.nn.sigmoid(dense(x_ang, asw, asb))
  x_sigs = jnp.abs(dense(dense(dense(z, s0w, s0b), s1w, s1b), s2w, s2b)) + EPS

  o_ref[:, 0:1] = x_loge
  o_ref[:, 1:2] = zeniazi[:, 0:1] * jnp.float32(jnp.pi)
  o_ref[:, 2:3] = zeniazi[:, 1:2] * jnp.float32(2.0 * jnp.pi)
  o_ref[:, 3:5] = x_sigs


def _decode(z, p):
  args = [z]
  for li in range(3):
    args += [p['dec%d_w' % li], p['dec%d_b' % li][None, :],
             p['bn%d_gamma' % li][None, :], p['bn%d_beta' % li][None, :],
             p['bn%d_mean' % li][None, :], p['bn%d_var' % li][None, :]]
  for name in ('loge0', 'loge1', 'loge_out',
               'ang0', 'ang1', 'ang_out', 'ang_scale',
               'sig0', 'sig1', 'sig_out'):
    args += [p[name + '_w'], p[name + '_b'][None, :]]
  return pl.pallas_call(
      _dec_body,
      out_shape=jax.ShapeDtypeStruct((G, 5), jnp.float32),
  )(*args)


def kernel(x, edge_index, segment_ids, params):
  pad_col = (jnp.arange(_EPAD, dtype=jnp.int32) * 37) % N
  pad_row = N + (jnp.arange(_EPAD, dtype=jnp.int32) % 8)
  row2 = jnp.concatenate([edge_index[0], pad_row]).reshape(_ROWS, _LANE)
  col2 = jnp.concatenate([edge_index[1], pad_col]).reshape(_ROWS, _LANE)
  seg2 = segment_ids[:, None]
  zero_h = jnp.zeros((_NACC, H), jnp.float32)
  zero_2h = jnp.zeros((_NACC, 2 * H), jnp.float32)

  hw0 = _matmul(x, params['gcn0_w'])
  agg0 = _edge_agg(H)(hw0, col2, row2, zero_h)
  hw1 = _relu_matmul(agg0, params['gcn0_b'][None, :], params['gcn1_w'])
  agg1 = _edge_agg(2 * H)(hw1, col2, row2, zero_2h)
  z = _pool(x, agg1, params['gcn1_b'][None, :], seg2)
  return _decode(z, params)


# speedup vs baseline: 8.7255x; 1.2087x over previous
import functools

import jax
import jax.numpy as jnp
from jax import lax
from jax.experimental import pallas as pl
from jax.experimental.pallas import tpu as pltpu
from jax.experimental.pallas import tpu_sc as plsc

N = 10000
E = 320000
D = 128
H = 64
G = 64
BN_EPS = 1e-3
EPS = 1e-5

_NC = 2
_NS = 16
_LANE = 64
_NW = _NC * _NS
_RPW = 160
_ROWS = _RPW * _NW
_EPAD = _ROWS * _LANE - E
_NACC = N + 8
_TPT = 632
_TPT_LAST = _NACC - 15 * _TPT
_WB_LAST = N - 15 * _TPT

_BLK = 2000
_NBLK = N // _BLK


def _make_edge_agg(W):
  mesh = plsc.VectorSubcoreMesh(core_axis_name="c", subcore_axis_name="s",
                                num_cores=_NC, num_subcores=_NS)

  @functools.partial(
      pl.kernel,
      mesh=mesh,
      out_type=jax.ShapeDtypeStruct((_NC, N, W), jnp.float32),
      scratch_types=[
          pltpu.VMEM((_RPW, _LANE), jnp.int32),
          pltpu.VMEM((_RPW, _LANE), jnp.int32),
          pltpu.VMEM((2, _LANE, W), jnp.float32),
          pltpu.VMEM_SHARED((_NACC, W), jnp.float32),
          pltpu.SemaphoreType.DMA,
      ],
      name="edge_agg_w%d" % W,
      compiler_params=pltpu.CompilerParams(use_tc_tiling_on_sc=False),
  )
  def edge_agg(hw_hbm, col_hbm, row_hbm, zero_hbm, out_hbm,
               col_v, row_v, gbuf, acc, sem):
    c = lax.axis_index("c")
    s = lax.axis_index("s")
    wid = c * _NS + s

    @pl.when(s < _NS - 1)
    def _():
      pltpu.sync_copy(zero_hbm.at[pl.ds(s * _TPT, _TPT)],
                      acc.at[pl.ds(s * _TPT, _TPT)])

    @pl.when(s == _NS - 1)
    def _():
      pltpu.sync_copy(zero_hbm.at[pl.ds(15 * _TPT, _TPT_LAST)],
                      acc.at[pl.ds(15 * _TPT, _TPT_LAST)])

    base = wid * _RPW
    pltpu.sync_copy(col_hbm.at[pl.ds(base, _RPW)], col_v)
    pltpu.sync_copy(row_hbm.at[pl.ds(base, _RPW)], row_v)

    plsc.subcore_barrier()

    pltpu.async_copy(hw_hbm.at[col_v.at[0]], gbuf.at[0], sem)

    def body(i, carry):
      p = lax.rem(i, 2)
      pltpu.make_async_copy(hw_hbm.at[col_v.at[i]], gbuf.at[p], sem).wait()

      @pl.when(i + 1 < _RPW)
      def _():
        pltpu.async_copy(hw_hbm.at[col_v.at[i + 1]], gbuf.at[1 - p], sem)

      pltpu.sync_copy(gbuf.at[p], acc.at[row_v.at[i]], add=True)
      return carry

    lax.fori_loop(0, _RPW, body, 0)

    plsc.subcore_barrier()

    @pl.when(s < _NS - 1)
    def _():
      pltpu.sync_copy(acc.at[pl.ds(s * _TPT, _TPT)],
                      out_hbm.at[c, pl.ds(s * _TPT, _TPT)])

    @pl.when(s == _NS - 1)
    def _():
      pltpu.sync_copy(acc.at[pl.ds(15 * _TPT, _WB_LAST)],
                      out_hbm.at[c, pl.ds(15 * _TPT, _WB_LAST)])

  return edge_agg


@functools.lru_cache(maxsize=None)
def _edge_agg(W):
  return _make_edge_agg(W)


def _mm_body(x_ref, w_ref, o_ref):
  o_ref[...] = jnp.dot(x_ref[...], w_ref[...],
                       preferred_element_type=jnp.float32)


def _matmul(x, w):
  m, k = x.shape
  _, n = w.shape
  return pl.pallas_call(
      _mm_body,
      grid=(_NBLK,),
      in_specs=[pl.BlockSpec((_BLK, k), lambda i: (i, 0)),
                pl.BlockSpec((k, n), lambda i: (0, 0))],
      out_specs=pl.BlockSpec((_BLK, n), lambda i: (i, 0)),
      out_shape=jax.ShapeDtypeStruct((m, n), jnp.float32),
  )(x, w)


def _relu_mm_body(a_ref, b_ref, w_ref, o_ref):
  h = jnp.maximum(a_ref[0] + a_ref[1] + b_ref[...], 0.0)
  o_ref[...] = jnp.dot(h, w_ref[...], preferred_element_type=jnp.float32)


def _relu_matmul(agg, b, w):
  k = agg.shape[2]
  n = w.shape[1]
  return pl.pallas_call(
      _relu_mm_body,
      grid=(_NBLK,),
      in_specs=[pl.BlockSpec((_NC, _BLK, k), lambda i: (0, i, 0)),
                pl.BlockSpec((1, k), lambda i: (0, 0)),
                pl.BlockSpec((k, n), lambda i: (0, 0))],
      out_specs=pl.BlockSpec((_BLK, n), lambda i: (i, 0)),
      out_shape=jax.ShapeDtypeStruct((N, n), jnp.float32),
  )(agg, b, w)


def _pool_body(x_ref, agg_ref, b_ref, seg_ref, z_ref,
               cnt, xsum, xsq, xmax, xmin, hmax, hsum):
  i = pl.program_id(0)

  @pl.when(i == 0)
  def _():
    cnt[...] = jnp.zeros_like(cnt)
    xsum[...] = jnp.zeros_like(xsum)
    xsq[...] = jnp.zeros_like(xsq)
    hsum[...] = jnp.zeros_like(hsum)
    xmax[...] = jnp.full_like(xmax, -jnp.inf)
    xmin[...] = jnp.full_like(xmin, jnp.inf)
    hmax[...] = jnp.full_like(hmax, -jnp.inf)

  x = x_ref[...]
  h = jnp.maximum(agg_ref[0] + agg_ref[1] + b_ref[...], 0.0)
  seg = seg_ref[...]
  gio = lax.broadcasted_iota(jnp.int32, (_BLK, G), 1)
  onehot = (seg == gio).astype(jnp.float32)
  ones = jnp.ones((_BLK, D), jnp.float32)
  dn = (((0,), (0,)), ((), ()))
  cnt[...] += lax.dot_general(onehot, ones, dn,
                              preferred_element_type=jnp.float32)
  xsum[...] += lax.dot_general(onehot, x, dn,
                               preferred_element_type=jnp.float32)
  xsq[...] += lax.dot_general(onehot, x * x, dn,
                              preferred_element_type=jnp.float32)
  hsum[...] += lax.dot_general(onehot, h, dn,
                               preferred_element_type=jnp.float32)

  g_lo = seg_ref[0, 0]
  g_hi = seg_ref[_BLK - 1, 0]
  rio = lax.broadcasted_iota(jnp.int32, (G, 1), 0)

  def gbody(g, carry):
    m = seg == g
    mx = jnp.max(jnp.where(m, x, -jnp.inf), axis=0, keepdims=True)
    mn = jnp.min(jnp.where(m, x, jnp.inf), axis=0, keepdims=True)
    mh = jnp.max(jnp.where(m, h, -jnp.inf), axis=0, keepdims=True)
    rs = rio == g
    xmax[...] = jnp.where(rs, jnp.maximum(xmax[...], mx), xmax[...])
    xmin[...] = jnp.where(rs, jnp.minimum(xmin[...], mn), xmin[...])
    hmax[...] = jnp.where(rs, jnp.maximum(hmax[...], mh), hmax[...])
    return carry

  lax.fori_loop(g_lo, g_hi + 1, gbody, 0)

  @pl.when(i == _NBLK - 1)
  def _():
    c = jnp.maximum(cnt[...], 1.0)
    avg = xsum[...] / c
    z_ref[:, 0:128] = hmax[...]
    z_ref[:, 128:256] = hsum[...] / c
    z_ref[:, 256:384] = hsum[...]
    z_ref[:, 384:512] = avg
    z_ref[:, 512:640] = jnp.abs(xsq[...] / c - avg * avg)
    z_ref[:, 640:768] = xmax[...]
    z_ref[:, 768:896] = xmin[...]


def _pool(x, agg, b, seg2):
  return pl.pallas_call(
      _pool_body,
      grid=(_NBLK,),
      in_specs=[pl.BlockSpec((_BLK, D), lambda i: (i, 0)),
                pl.BlockSpec((_NC, _BLK, D), lambda i: (0, i, 0)),
                pl.BlockSpec((1, D), lambda i: (0, 0)),
                pl.BlockSpec((_BLK, 1), lambda i: (i, 0))],
      out_specs=pl.BlockSpec((G, 896), lambda i: (0, 0)),
      out_shape=jax.ShapeDtypeStruct((G, 896), jnp.float32),
      scratch_shapes=[pltpu.VMEM((G, D), jnp.float32)] * 7,
  )(x, agg, b, seg2)


def _dec_body(z_ref, *refs):
  (d0w, d0b, g0, be0, m0, v0,
   d1w, d1b, g1, be1, m1, v1,
   d2w, d2b, g2, be2, m2, v2,
   l0w, l0b, l1w, l1b, l2w, l2b,
   a0w, a0b, a1w, a1b, a2w, a2b, asw, asb,
   s0w, s0b, s1w, s1b, s2w, s2b, o_ref) = refs

  z = z_ref[...]
  for (w, b, gm, bt, mu, vr) in ((d0w, d0b, g0, be0, m0, v0),
                                 (d1w, d1b, g1, be1, m1, v1),
                                 (d2w, d2b, g2, be2, m2, v2)):
    z = jnp.dot(z, w[...], preferred_element_type=jnp.float32) + b[...]
    z = jnp.where(z >= 0, z, 0.15 * z)
    z = ((z - mu[...]) / jnp.sqrt(vr[...] + BN_EPS)) * gm[...] + bt[...]

  def dense(t, w, b):
    return jnp.dot(t, w[...], preferred_element_type=jnp.float32) + b[...]

  x_loge = dense(dense(dense(z, l0w, l0b), l1w, l1b), l2w, l2b)
  x_ang = dense(dense(dense(z, a0w, a0b), a1w, a1b), a2w, a2b)
  zeniazi = jax.nn.sigmoid(dense(x_ang, asw, asb))
  x_sigs = jnp.abs(dense(dense(dense(z, s0w, s0b), s1w, s1b), s2w, s2b)) + EPS

  o_ref[:, 0:1] = x_loge
  o_ref[:, 1:2] = zeniazi[:, 0:1] * jnp.float32(jnp.pi)
  o_ref[:, 2:3] = zeniazi[:, 1:2] * jnp.float32(2.0 * jnp.pi)
  o_ref[:, 3:5] = x_sigs


def _decode(z, p):
  args = [z]
  for li in range(3):
    args += [p['dec%d_w' % li], p['dec%d_b' % li][None, :],
             p['bn%d_gamma' % li][None, :], p['bn%d_beta' % li][None, :],
             p['bn%d_mean' % li][None, :], p['bn%d_var' % li][None, :]]
  for name in ('loge0', 'loge1', 'loge_out',
               'ang0', 'ang1', 'ang_out', 'ang_scale',
               'sig0', 'sig1', 'sig_out'):
    args += [p[name + '_w'], p[name + '_b'][None, :]]
  return pl.pallas_call(
      _dec_body,
      out_shape=jax.ShapeDtypeStruct((G, 5), jnp.float32),
  )(*args)


def kernel(x, edge_index, segment_ids, params):
  pad_col = (jnp.arange(_EPAD, dtype=jnp.int32) * 37) % N
  pad_row = N + (jnp.arange(_EPAD, dtype=jnp.int32) % 8)
  row2 = jnp.concatenate([edge_index[0], pad_row]).reshape(_ROWS, _LANE)
  col2 = jnp.concatenate([edge_index[1], pad_col]).reshape(_ROWS, _LANE)
  seg2 = segment_ids[:, None]
  zero_h = jnp.zeros((_NACC, H), jnp.float32)
  zero_2h = jnp.zeros((_NACC, 2 * H), jnp.float32)

  hw0 = _matmul(x, params['gcn0_w'])
  agg0 = _edge_agg(H)(hw0, col2, row2, zero_h)
  hw1 = _relu_matmul(agg0, params['gcn0_b'][None, :], params['gcn1_w'])
  agg1 = _edge_agg(2 * H)(hw1, col2, row2, zero_2h)
  z = _pool(x, agg1, params['gcn1_b'][None, :], seg2)
  return _decode(z, params)


# speedup vs baseline: 12.5068x; 1.4334x over previous
import functools

import jax
import jax.numpy as jnp
from jax import lax
from jax.experimental import pallas as pl
from jax.experimental.pallas import tpu as pltpu
from jax.experimental.pallas import tpu_sc as plsc

N = 10000
E = 320000
D = 128
H = 64
G = 64
BN_EPS = 1e-3
EPS = 1e-5

_NC = 2
_NS = 16
_LANE = 64
_NW = _NC * _NS
_RPW = 160
_ROWS = _RPW * _NW
_EPAD = _ROWS * _LANE - E
_NACC = N + 8
_TPT = 632
_TPT_LAST = _NACC - 15 * _TPT
_WB_LAST = N - 15 * _TPT

_BLK = 2000
_NBLK = N // _BLK


def _make_edge_agg(W):
  mesh = plsc.VectorSubcoreMesh(core_axis_name="c", subcore_axis_name="s",
                                num_cores=_NC, num_subcores=_NS)

  @functools.partial(
      pl.kernel,
      mesh=mesh,
      out_type=jax.ShapeDtypeStruct((_NC, N, W), jnp.float32),
      scratch_types=[
          pltpu.VMEM((_RPW, _LANE), jnp.int32),
          pltpu.VMEM((_RPW, _LANE), jnp.int32),
          pltpu.VMEM((3, _LANE, W), jnp.float32),
          pltpu.VMEM_SHARED((_NACC, W), jnp.float32),
          pltpu.SemaphoreType.DMA,
          pltpu.SemaphoreType.DMA,
      ],
      name="edge_agg_w%d" % W,
      compiler_params=pltpu.CompilerParams(use_tc_tiling_on_sc=False),
  )
  def edge_agg(hw_hbm, col_hbm, row_hbm, zero_hbm, out_hbm,
               col_v, row_v, gbuf, acc, sem, sem2):
    c = lax.axis_index("c")
    s = lax.axis_index("s")
    wid = c * _NS + s

    @pl.when(s < _NS - 1)
    def _():
      pltpu.sync_copy(zero_hbm.at[pl.ds(s * _TPT, _TPT)],
                      acc.at[pl.ds(s * _TPT, _TPT)])

    @pl.when(s == _NS - 1)
    def _():
      pltpu.sync_copy(zero_hbm.at[pl.ds(15 * _TPT, _TPT_LAST)],
                      acc.at[pl.ds(15 * _TPT, _TPT_LAST)])

    base = wid * _RPW
    pltpu.sync_copy(col_hbm.at[pl.ds(base, _RPW)], col_v)
    pltpu.sync_copy(row_hbm.at[pl.ds(base, _RPW)], row_v)

    plsc.subcore_barrier()

    pltpu.async_copy(hw_hbm.at[col_v.at[0]], gbuf.at[0], sem)
    pltpu.async_copy(hw_hbm.at[col_v.at[1]], gbuf.at[1], sem)

    def body(i, carry):
      p = lax.rem(i, 3)
      pltpu.make_async_copy(hw_hbm.at[col_v.at[i]], gbuf.at[p], sem).wait()
      pltpu.async_copy(gbuf.at[p], acc.at[row_v.at[i]], sem2, add=True)

      @pl.when(i >= 1)
      def _():
        pm1 = lax.rem(i + 2, 3)
        pltpu.make_async_copy(gbuf.at[pm1], acc.at[row_v.at[i - 1]],
                              sem2).wait()

      @pl.when(i + 2 < _RPW)
      def _():
        pltpu.async_copy(hw_hbm.at[col_v.at[i + 2]], gbuf.at[lax.rem(i + 2, 3)],
                         sem)

      return carry

    lax.fori_loop(0, _RPW, body, 0)
    pltpu.make_async_copy(gbuf.at[lax.rem(_RPW - 1, 3)],
                          acc.at[row_v.at[_RPW - 1]], sem2).wait()

    plsc.subcore_barrier()

    @pl.when(s < _NS - 1)
    def _():
      pltpu.sync_copy(acc.at[pl.ds(s * _TPT, _TPT)],
                      out_hbm.at[c, pl.ds(s * _TPT, _TPT)])

    @pl.when(s == _NS - 1)
    def _():
      pltpu.sync_copy(acc.at[pl.ds(15 * _TPT, _WB_LAST)],
                      out_hbm.at[c, pl.ds(15 * _TPT, _WB_LAST)])

  return edge_agg


@functools.lru_cache(maxsize=None)
def _edge_agg(W):
  return _make_edge_agg(W)


def _mm_body(x_ref, w_ref, o_ref):
  o_ref[...] = jnp.dot(x_ref[...], w_ref[...],
                       preferred_element_type=jnp.float32)


def _matmul(x, w):
  m, k = x.shape
  _, n = w.shape
  return pl.pallas_call(
      _mm_body,
      grid=(_NBLK,),
      in_specs=[pl.BlockSpec((_BLK, k), lambda i: (i, 0)),
                pl.BlockSpec((k, n), lambda i: (0, 0))],
      out_specs=pl.BlockSpec((_BLK, n), lambda i: (i, 0)),
      out_shape=jax.ShapeDtypeStruct((m, n), jnp.float32),
  )(x, w)


def _relu_mm_body(a_ref, b_ref, w_ref, o_ref):
  h = jnp.maximum(a_ref[0] + a_ref[1] + b_ref[...], 0.0)
  o_ref[...] = jnp.dot(h, w_ref[...], preferred_element_type=jnp.float32)


def _relu_matmul(agg, b, w):
  k = agg.shape[2]
  n = w.shape[1]
  return pl.pallas_call(
      _relu_mm_body,
      grid=(_NBLK,),
      in_specs=[pl.BlockSpec((_NC, _BLK, k), lambda i: (0, i, 0)),
                pl.BlockSpec((1, k), lambda i: (0, 0)),
                pl.BlockSpec((k, n), lambda i: (0, 0))],
      out_specs=pl.BlockSpec((_BLK, n), lambda i: (i, 0)),
      out_shape=jax.ShapeDtypeStruct((N, n), jnp.float32),
  )(agg, b, w)


def _pool_body(x_ref, agg_ref, b_ref, seg_ref, z_ref,
               cnt, xsum, xsq, xmax, xmin, hmax, hsum):
  i = pl.program_id(0)

  @pl.when(i == 0)
  def _():
    cnt[...] = jnp.zeros_like(cnt)
    xsum[...] = jnp.zeros_like(xsum)
    xsq[...] = jnp.zeros_like(xsq)
    hsum[...] = jnp.zeros_like(hsum)
    xmax[...] = jnp.full_like(xmax, -jnp.inf)
    xmin[...] = jnp.full_like(xmin, jnp.inf)
    hmax[...] = jnp.full_like(hmax, -jnp.inf)

  x = x_ref[...]
  h = jnp.maximum(agg_ref[0] + agg_ref[1] + b_ref[...], 0.0)
  seg = seg_ref[...]
  gio = lax.broadcasted_iota(jnp.int32, (_BLK, G), 1)
  onehot = (seg == gio).astype(jnp.float32)
  ones = jnp.ones((_BLK, D), jnp.float32)
  dn = (((0,), (0,)), ((), ()))
  cnt[...] += lax.dot_general(onehot, ones, dn,
                              preferred_element_type=jnp.float32)
  xsum[...] += lax.dot_general(onehot, x, dn,
                               preferred_element_type=jnp.float32)
  xsq[...] += lax.dot_general(onehot, x * x, dn,
                              preferred_element_type=jnp.float32)
  hsum[...] += lax.dot_general(onehot, h, dn,
                               preferred_element_type=jnp.float32)

  g_lo = seg_ref[0, 0]
  g_hi = seg_ref[_BLK - 1, 0]
  rio = lax.broadcasted_iota(jnp.int32, (G, 1), 0)

  def gbody(g, carry):
    m = seg == g
    mx = jnp.max(jnp.where(m, x, -jnp.inf), axis=0, keepdims=True)
    mn = jnp.min(jnp.where(m, x, jnp.inf), axis=0, keepdims=True)
    mh = jnp.max(jnp.where(m, h, -jnp.inf), axis=0, keepdims=True)
    rs = rio == g
    xmax[...] = jnp.where(rs, jnp.maximum(xmax[...], mx), xmax[...])
    xmin[...] = jnp.where(rs, jnp.minimum(xmin[...], mn), xmin[...])
    hmax[...] = jnp.where(rs, jnp.maximum(hmax[...], mh), hmax[...])
    return carry

  lax.fori_loop(g_lo, g_hi + 1, gbody, 0)

  @pl.when(i == _NBLK - 1)
  def _():
    c = jnp.maximum(cnt[...], 1.0)
    avg = xsum[...] / c
    z_ref[:, 0:128] = hmax[...]
    z_ref[:, 128:256] = hsum[...] / c
    z_ref[:, 256:384] = hsum[...]
    z_ref[:, 384:512] = avg
    z_ref[:, 512:640] = jnp.abs(xsq[...] / c - avg * avg)
    z_ref[:, 640:768] = xmax[...]
    z_ref[:, 768:896] = xmin[...]


def _pool(x, agg, b, seg2):
  return pl.pallas_call(
      _pool_body,
      grid=(_NBLK,),
      in_specs=[pl.BlockSpec((_BLK, D), lambda i: (i, 0)),
                pl.BlockSpec((_NC, _BLK, D), lambda i: (0, i, 0)),
                pl.BlockSpec((1, D), lambda i: (0, 0)),
                pl.BlockSpec((_BLK, 1), lambda i: (i, 0))],
      out_specs=pl.BlockSpec((G, 896), lambda i: (0, 0)),
      out_shape=jax.ShapeDtypeStruct((G, 896), jnp.float32),
      scratch_shapes=[pltpu.VMEM((G, D), jnp.float32)] * 7,
  )(x, agg, b, seg2)


def _dec_body(z_ref, *refs):
  (d0w, d0b, g0, be0, m0, v0,
   d1w, d1b, g1, be1, m1, v1,
   d2w, d2b, g2, be2, m2, v2,
   l0w, l0b, l1w, l1b, l2w, l2b,
   a0w, a0b, a1w, a1b, a2w, a2b, asw, asb,
   s0w, s0b, s1w, s1b, s2w, s2b, o_ref) = refs

  z = z_ref[...]
  for (w, b, gm, bt, mu, vr) in ((d0w, d0b, g0, be0, m0, v0),
                                 (d1w, d1b, g1, be1, m1, v1),
                                 (d2w, d2b, g2, be2, m2, v2)):
    z = jnp.dot(z, w[...], preferred_element_type=jnp.float32) + b[...]
    z = jnp.where(z >= 0, z, 0.15 * z)
    z = ((z - mu[...]) / jnp.sqrt(vr[...] + BN_EPS)) * gm[...] + bt[...]

  def dense(t, w, b):
    return jnp.dot(t, w[...], preferred_element_type=jnp.float32) + b[...]

  x_loge = dense(dense(dense(z, l0w, l0b), l1w, l1b), l2w, l2b)
  x_ang = dense(dense(dense(z, a0w, a0b), a1w, a1b), a2w, a2b)
  zeniazi = jax.nn.sigmoid(dense(x_ang, asw, asb))
  x_sigs = jnp.abs(dense(dense(dense(z, s0w, s0b), s1w, s1b), s2w, s2b)) + EPS

  o_ref[:, 0:1] = x_loge
  o_ref[:, 1:2] = zeniazi[:, 0:1] * jnp.float32(jnp.pi)
  o_ref[:, 2:3] = zeniazi[:, 1:2] * jnp.float32(2.0 * jnp.pi)
  o_ref[:, 3:5] = x_sigs


def _decode(z, p):
  args = [z]
  for li in range(3):
    args += [p['dec%d_w' % li], p['dec%d_b' % li][None, :],
             p['bn%d_gamma' % li][None, :], p['bn%d_beta' % li][None, :],
             p['bn%d_mean' % li][None, :], p['bn%d_var' % li][None, :]]
  for name in ('loge0', 'loge1', 'loge_out',
               'ang0', 'ang1', 'ang_out', 'ang_scale',
               'sig0', 'sig1', 'sig_out'):
    args += [p[name + '_w'], p[name + '_b'][None, :]]
  return pl.pallas_call(
      _dec_body,
      out_shape=jax.ShapeDtypeStruct((G, 5), jnp.float32),
  )(*args)


def kernel(x, edge_index, segment_ids, params):
  pad_col = (jnp.arange(_EPAD, dtype=jnp.int32) * 37) % N
  pad_row = N + (jnp.arange(_EPAD, dtype=jnp.int32) % 8)
  row2 = jnp.concatenate([edge_index[0], pad_row]).reshape(_ROWS, _LANE)
  col2 = jnp.concatenate([edge_index[1], pad_col]).reshape(_ROWS, _LANE)
  seg2 = segment_ids[:, None]
  zero_h = jnp.zeros((_NACC, H), jnp.float32)
  zero_2h = jnp.zeros((_NACC, 2 * H), jnp.float32)

  hw0 = _matmul(x, params['gcn0_w'])
  agg0 = _edge_agg(H)(hw0, col2, row2, zero_h)
  hw1 = _relu_matmul(agg0, params['gcn0_b'][None, :], params['gcn1_w'])
  agg1 = _edge_agg(2 * H)(hw1, col2, row2, zero_2h)
  z = _pool(x, agg1, params['gcn1_b'][None, :], seg2)
  return _decode(z, params)
